# baseline (speedup 1.0000x reference)
import functools
import math

import jax
import jax.numpy as jnp
from jax import lax
from jax.experimental import pallas as pl
from jax.experimental.pallas import tpu as pltpu
from jax.experimental.pallas import tpu_sc as plsc

LANES = 16
CHUNK = 32
NSLOT = 4
NW = 32
BM = 400


def _dense_body(x_ref, ws_ref, bs_ref, wr_ref, br_ref, wa_ref, ba_ref,
                self_ref, yc_ref, av_ref):
    xb = x_ref[...]
    self_ref[...] = (
        jnp.dot(xb, ws_ref[...], preferred_element_type=jnp.float32)
        + bs_ref[...])
    yc_ref[...] = (
        jnp.dot(xb, wr_ref[...], preferred_element_type=jnp.float32)
        + br_ref[...])
    av_ref[...] = (
        jnp.dot(xb, wa_ref[...], preferred_element_type=jnp.float32)
        + ba_ref[...])


def _dense_stage(x, ws, bs, wr, br, wa, ba):
    n, d = x.shape
    rd = wr.shape[1]
    grid = n // BM
    return pl.pallas_call(
        _dense_body,
        grid=(grid,),
        in_specs=[
            pl.BlockSpec((BM, d), lambda i: (i, 0)),
            pl.BlockSpec((d, d), lambda i: (0, 0)),
            pl.BlockSpec((1, d), lambda i: (0, 0)),
            pl.BlockSpec((d, rd), lambda i: (0, 0)),
            pl.BlockSpec((1, rd), lambda i: (0, 0)),
            pl.BlockSpec((d, 8), lambda i: (0, 0)),
            pl.BlockSpec((1, 8), lambda i: (0, 0)),
        ],
        out_specs=[
            pl.BlockSpec((BM, d), lambda i: (i, 0)),
            pl.BlockSpec((BM, rd), lambda i: (i, 0)),
            pl.BlockSpec((BM, 8), lambda i: (i, 0)),
        ],
        out_shape=[
            jax.ShapeDtypeStruct((n, d), jnp.float32),
            jax.ShapeDtypeStruct((n, rd), jnp.float32),
            jax.ShapeDtypeStruct((n, 8), jnp.float32),
        ],
    )(x, ws, bs, wr, br, wa, ba)


def _sc_body(n_nodes, n_acc, r_rel, per_worker,
             y_hbm, asrc_hbm, atgt_hbm, epk_hbm, zeros_hbm,
             partial_hbm,
             acc, asrc_v, atgt_v, ebuf_v,
             g_v, tc_v, att_v, rows_v,
             esem0, esem1, esem2, esem3, gsem0, gsem1, gsem2, gsem3,
             ssem0, ssem1, ssem2, ssem3):
    esem = (esem0, esem1, esem2, esem3)
    gsem = (gsem0, gsem1, gsem2, gsem3)
    ssem = (ssem0, ssem1, ssem2, ssem3)
    cid = lax.axis_index("c")
    sid = lax.axis_index("s")
    wid = cid * 16 + sid
    stripe = n_acc // 16
    n_chunks = per_worker // CHUNK
    n_groups = n_chunks // NSLOT
    chunk_w = 3 * CHUNK
    base_w = wid * n_chunks

    pltpu.sync_copy(zeros_hbm.at[pl.ds(sid * stripe, stripe)],
                    acc.at[pl.ds(sid * stripe, stripe)])

    pltpu.sync_copy(asrc_hbm, asrc_v)
    pltpu.sync_copy(atgt_hbm, atgt_v)
    plsc.subcore_barrier()

    def fire_e(c, s):
        pltpu.async_copy(epk_hbm.at[pl.ds((base_w + c) * chunk_w, chunk_w)],
                         ebuf_v.at[s], esem[s])

    def wait_e(c, s):
        pltpu.make_async_copy(
            epk_hbm.at[pl.ds((base_w + c) * chunk_w, chunk_w)],
            ebuf_v.at[s], esem[s]).wait()

    def prep(s):
        for k in range(CHUNK // LANES):
            s16 = ebuf_v[s, pl.ds(k * LANES, LANES)]
            t16 = ebuf_v[s, pl.ds(CHUNK + k * LANES, LANES)]
            ty16 = ebuf_v[s, pl.ds(2 * CHUNK + k * LANES, LANES)]
            sl = pl.ds(k * LANES, LANES)
            g_v[s, sl] = s16 * r_rel + ty16
            tc_v[s, sl] = t16
            za = plsc.load_gather(asrc_v, [s16])
            zb = plsc.load_gather(atgt_v, [t16])
            att_v[pl.ds(s * CHUNK + k * LANES, LANES)] = (
                1.0 / (1.0 + jnp.exp(-(za + zb))))

    def fire_g(s):
        pltpu.async_copy(y_hbm.at[g_v.at[s]], rows_v.at[s], gsem[s])

    def wait_g(s):
        pltpu.make_async_copy(y_hbm.at[g_v.at[s]], rows_v.at[s],
                              gsem[s]).wait()

    def scale(s):
        def scale_body(e, carry):
            for u in range(2):
                eu = e * 2 + u
                a = plsc.load_gather(
                    att_v, [jnp.full((LANES,), s * CHUNK + eu, jnp.int32)])
                for dd in range(128 // LANES):
                    sl = pl.ds(dd * LANES, LANES)
                    rows_v[s, eu, sl] = rows_v[s, eu, sl] * a
            return carry

        lax.fori_loop(0, CHUNK // 2, scale_body, 0)

    def fire_w(s):
        pltpu.async_copy(rows_v.at[s], acc.at[tc_v.at[s]], ssem[s], add=True)

    def wait_w(s):
        pltpu.make_async_copy(rows_v.at[s], acc.at[tc_v.at[s]],
                              ssem[s]).wait()

    for s in range(NSLOT):
        fire_e(s, s)
    wait_e(0, 0)
    prep(0)
    fire_g(0)
    wait_e(1, 1)
    prep(1)
    fire_g(1)

    def group_body(j, carry):
        for s in range(NSLOT):
            c = j * NSLOT + s
            ns2 = (s + 2) % NSLOT
            if s >= 2:
                wait_w(ns2)
            else:
                @pl.when(j >= 1)
                def _retire(ns2=ns2):
                    wait_w(ns2)
            if s >= 2:
                @pl.when(j < n_groups - 1)
                def _stage(c=c, ns2=ns2):
                    wait_e(c + 2, ns2)
                    prep(ns2)
                    fire_g(ns2)
            else:
                wait_e(c + 2, ns2)
                prep(ns2)
                fire_g(ns2)
            @pl.when(j < n_groups - 1)
            def _prefetch(c=c, s=s):
                fire_e(c + NSLOT, s)
            wait_g(s)
            scale(s)
            fire_w(s)
        return carry

    lax.fori_loop(0, n_groups, group_body, 0)
    wait_w((n_chunks - 2) % NSLOT)
    wait_w((n_chunks - 1) % NSLOT)
    plsc.subcore_barrier()

    dst0 = cid * n_acc + sid * stripe
    pltpu.sync_copy(acc.at[pl.ds(sid * stripe, stripe)],
                    partial_hbm.at[pl.ds(dst0, stripe)])


def _sparse_stage(y, a_src, a_tgt, epk, zeros, n_nodes, n_acc,
                  r_rel, per_worker):
    d = y.shape[1]
    mesh = plsc.VectorSubcoreMesh(core_axis_name="c", subcore_axis_name="s",
                                  num_cores=2, num_subcores=16)
    body = functools.partial(_sc_body, n_nodes, n_acc, r_rel, per_worker)
    return pl.kernel(
        body,
        out_type=jax.ShapeDtypeStruct((2 * n_acc, d), jnp.float32),
        mesh=mesh,
        scratch_types=[
            pltpu.VMEM_SHARED((n_acc, d), jnp.float32),
            pltpu.VMEM((n_nodes,), jnp.float32),
            pltpu.VMEM((n_nodes,), jnp.float32),
            pltpu.VMEM((NSLOT, 3 * CHUNK), jnp.int32),
            pltpu.VMEM((NSLOT, CHUNK), jnp.int32),
            pltpu.VMEM((NSLOT, CHUNK), jnp.int32),
            pltpu.VMEM((NSLOT * CHUNK,), jnp.float32),
            pltpu.VMEM((NSLOT, CHUNK, d), jnp.float32),
        ] + [pltpu.SemaphoreType.DMA] * 12,
        compiler_params=pltpu.CompilerParams(needs_layout_passes=False),
    )(y, a_src, a_tgt, epk, zeros)


def _add_body(a_ref, b_ref, c_ref, o_ref):
    o_ref[...] = a_ref[...] + b_ref[...] + c_ref[...]


def _final_add(self_out, p0, p1, n, d):
    grid = n // BM
    return pl.pallas_call(
        _add_body,
        grid=(grid,),
        in_specs=[
            pl.BlockSpec((BM, d), lambda i: (i, 0)),
            pl.BlockSpec((BM, d), lambda i: (i, 0)),
            pl.BlockSpec((BM, d), lambda i: (i, 0)),
        ],
        out_specs=pl.BlockSpec((BM, d), lambda i: (i, 0)),
        out_shape=jax.ShapeDtypeStruct((n, d), jnp.float32),
    )(self_out, p0, p1)


def kernel(x, edge_index, edge_types, rel_W, rel_b, self_W, self_b,
           att_W, att_b):
    n, d = x.shape
    r = rel_W.shape[0]
    e = edge_index.shape[1]

    ws = self_W.T
    bs = self_b.reshape(1, d)
    wr = jnp.transpose(rel_W, (2, 0, 1)).reshape(d, r * d)
    br = rel_b.reshape(1, r * d)
    wa = jnp.zeros((d, 8), jnp.float32)
    wa = wa.at[:, 0].set(att_W[0, :d]).at[:, 1].set(att_W[0, d:])
    ba = jnp.zeros((1, 8), jnp.float32).at[0, 0].set(att_b[0])

    self_out, yc, av = _dense_stage(x, ws, bs, wr, br, wa, ba)
    y = yc.reshape(n * r, d)
    a_src = av[:, 0]
    a_tgt = av[:, 1]

    ring = NSLOT * CHUNK
    per_worker = -(-e // (NW * ring)) * ring
    ep = per_worker * NW
    n_acc = -(-(n + LANES) // 128) * 128
    pad = ep - e
    src = edge_index[0].astype(jnp.int32)
    tgt = edge_index[1].astype(jnp.int32)
    typ = edge_types.astype(jnp.int32)
    if pad:
        src = jnp.concatenate([src, jnp.zeros((pad,), jnp.int32)])
        tgt = jnp.concatenate([tgt, jnp.full((pad,), n, jnp.int32)])
        typ = jnp.concatenate([typ, jnp.zeros((pad,), jnp.int32)])
    epk = jnp.concatenate(
        [src.reshape(-1, 1, CHUNK), tgt.reshape(-1, 1, CHUNK),
         typ.reshape(-1, 1, CHUNK)], axis=1).reshape(-1)
    zeros = jnp.zeros((n_acc, d), jnp.float32)

    partial = _sparse_stage(y, a_src, a_tgt, epk, zeros,
                            n, n_acc, r, per_worker)
    p0 = lax.slice(partial, (0, 0), (n, d))
    p1 = lax.slice(partial, (n_acc, 0), (n_acc + n, d))
    return _final_add(self_out, p0, p1, n, d)


# speedup vs baseline: 1.0915x; 1.0915x over previous
import functools
import math

import jax
import jax.numpy as jnp
from jax import lax
from jax.experimental import pallas as pl
from jax.experimental.pallas import tpu as pltpu
from jax.experimental.pallas import tpu_sc as plsc

LANES = 16
CHUNK = 32
NSLOT = 3
NW = 32
BM = 400


def _dense_body(x_ref, ws_ref, bs_ref, wr_ref, br_ref, wa_ref, ba_ref,
                self_ref, yc_ref, av_ref):
    xb = x_ref[...]
    self_ref[...] = (
        jnp.dot(xb, ws_ref[...], preferred_element_type=jnp.float32)
        + bs_ref[...])
    yc_ref[...] = (
        jnp.dot(xb, wr_ref[...], preferred_element_type=jnp.float32)
        + br_ref[...])
    av_ref[...] = (
        jnp.dot(xb, wa_ref[...], preferred_element_type=jnp.float32)
        + ba_ref[...])


def _dense_stage(x, ws, bs, wr, br, wa, ba):
    n, d = x.shape
    rd = wr.shape[1]
    grid = n // BM
    return pl.pallas_call(
        _dense_body,
        grid=(grid,),
        in_specs=[
            pl.BlockSpec((BM, d), lambda i: (i, 0)),
            pl.BlockSpec((d, d), lambda i: (0, 0)),
            pl.BlockSpec((1, d), lambda i: (0, 0)),
            pl.BlockSpec((d, rd), lambda i: (0, 0)),
            pl.BlockSpec((1, rd), lambda i: (0, 0)),
            pl.BlockSpec((d, 8), lambda i: (0, 0)),
            pl.BlockSpec((1, 8), lambda i: (0, 0)),
        ],
        out_specs=[
            pl.BlockSpec((BM, d), lambda i: (i, 0)),
            pl.BlockSpec((BM, rd), lambda i: (i, 0)),
            pl.BlockSpec((BM, 8), lambda i: (i, 0)),
        ],
        out_shape=[
            jax.ShapeDtypeStruct((n, d), jnp.float32),
            jax.ShapeDtypeStruct((n, rd), jnp.float32),
            jax.ShapeDtypeStruct((n, 8), jnp.float32),
        ],
    )(x, ws, bs, wr, br, wa, ba)


def _sc_body(n_nodes, n_acc, r_rel, per_worker,
             y_hbm, asrc_hbm, atgt_hbm, epk_hbm, zeros_hbm,
             partial_hbm,
             acc, asrc_v, atgt_v, ebuf_v,
             g_v, tc_v, att_v, rows_v,
             esem0, esem1, esem2, gsem0, gsem1, gsem2,
             ssem0, ssem1, ssem2):
    esem = (esem0, esem1, esem2)
    gsem = (gsem0, gsem1, gsem2)
    ssem = (ssem0, ssem1, ssem2)
    cid = lax.axis_index("c")
    sid = lax.axis_index("s")
    wid = cid * 16 + sid
    stripe = n_acc // 16
    n_chunks = per_worker // CHUNK
    n_groups = n_chunks // NSLOT
    chunk_w = 3 * CHUNK
    base_w = wid * n_chunks

    pltpu.sync_copy(zeros_hbm.at[pl.ds(sid * stripe, stripe)],
                    acc.at[pl.ds(sid * stripe, stripe)])

    pltpu.sync_copy(asrc_hbm, asrc_v)
    pltpu.sync_copy(atgt_hbm, atgt_v)
    plsc.subcore_barrier()

    def fire_e(c, s):
        pltpu.async_copy(epk_hbm.at[pl.ds((base_w + c) * chunk_w, chunk_w)],
                         ebuf_v.at[s], esem[s])

    def wait_e(c, s):
        pltpu.make_async_copy(
            epk_hbm.at[pl.ds((base_w + c) * chunk_w, chunk_w)],
            ebuf_v.at[s], esem[s]).wait()

    def prep(s):
        for k in range(CHUNK // LANES):
            s16 = ebuf_v[s, pl.ds(k * LANES, LANES)]
            t16 = ebuf_v[s, pl.ds(CHUNK + k * LANES, LANES)]
            ty16 = ebuf_v[s, pl.ds(2 * CHUNK + k * LANES, LANES)]
            sl = pl.ds(k * LANES, LANES)
            g_v[s, sl] = s16 * r_rel + ty16
            tc_v[s, sl] = t16
            za = plsc.load_gather(asrc_v, [s16])
            zb = plsc.load_gather(atgt_v, [t16])
            att_v[pl.ds(s * CHUNK + k * LANES, LANES)] = (
                1.0 / (1.0 + jnp.exp(-(za + zb))))

    def fire_g(s):
        pltpu.async_copy(y_hbm.at[g_v.at[s]], rows_v.at[s], gsem[s])

    def wait_g(s):
        pltpu.make_async_copy(y_hbm.at[g_v.at[s]], rows_v.at[s],
                              gsem[s]).wait()

    def scale(s):
        def scale_body(e, carry):
            for u in range(2):
                eu = e * 2 + u
                a = plsc.load_gather(
                    att_v, [jnp.full((LANES,), s * CHUNK + eu, jnp.int32)])
                for dd in range(128 // LANES):
                    sl = pl.ds(dd * LANES, LANES)
                    rows_v[s, eu, sl] = rows_v[s, eu, sl] * a
            return carry

        lax.fori_loop(0, CHUNK // 2, scale_body, 0)

    def fire_w(s):
        pltpu.async_copy(rows_v.at[s], acc.at[tc_v.at[s]], ssem[s], add=True)

    def wait_w(s):
        pltpu.make_async_copy(rows_v.at[s], acc.at[tc_v.at[s]],
                              ssem[s]).wait()

    for s in range(NSLOT):
        fire_e(s, s)
    wait_e(0, 0)
    prep(0)
    fire_g(0)

    def group_body(j, carry):
        for s in range(NSLOT):
            c = j * NSLOT + s
            ns = (s + 1) % NSLOT
            if s == 2:
                wait_w(ns)
            else:
                @pl.when(j >= 1)
                def _retire(ns=ns):
                    wait_w(ns)
            if s == 2:
                @pl.when(j < n_groups - 1)
                def _stage(c=c, ns=ns):
                    wait_e(c + 1, ns)
                    prep(ns)
                    fire_g(ns)
            else:
                wait_e(c + 1, ns)
                prep(ns)
                fire_g(ns)
            @pl.when(j < n_groups - 1)
            def _prefetch(c=c, s=s):
                fire_e(c + NSLOT, s)
            wait_g(s)
            scale(s)
            fire_w(s)
        return carry

    lax.fori_loop(0, n_groups, group_body, 0)
    wait_w((n_chunks - 2) % NSLOT)
    wait_w((n_chunks - 1) % NSLOT)
    plsc.subcore_barrier()

    dst0 = cid * n_acc + sid * stripe
    pltpu.sync_copy(acc.at[pl.ds(sid * stripe, stripe)],
                    partial_hbm.at[pl.ds(dst0, stripe)])


def _sparse_stage(y, a_src, a_tgt, epk, zeros, n_nodes, n_acc,
                  r_rel, per_worker):
    d = y.shape[1]
    mesh = plsc.VectorSubcoreMesh(core_axis_name="c", subcore_axis_name="s",
                                  num_cores=2, num_subcores=16)
    body = functools.partial(_sc_body, n_nodes, n_acc, r_rel, per_worker)
    return pl.kernel(
        body,
        out_type=jax.ShapeDtypeStruct((2 * n_acc, d), jnp.float32),
        mesh=mesh,
        scratch_types=[
            pltpu.VMEM_SHARED((n_acc, d), jnp.float32),
            pltpu.VMEM((n_nodes,), jnp.float32),
            pltpu.VMEM((n_nodes,), jnp.float32),
            pltpu.VMEM((NSLOT, 3 * CHUNK), jnp.int32),
            pltpu.VMEM((NSLOT, CHUNK), jnp.int32),
            pltpu.VMEM((NSLOT, CHUNK), jnp.int32),
            pltpu.VMEM((NSLOT * CHUNK,), jnp.float32),
            pltpu.VMEM((NSLOT, CHUNK, d), jnp.float32),
        ] + [pltpu.SemaphoreType.DMA] * 9,
        compiler_params=pltpu.CompilerParams(needs_layout_passes=False),
    )(y, a_src, a_tgt, epk, zeros)


def _add_body(a_ref, b_ref, c_ref, o_ref):
    o_ref[...] = a_ref[...] + b_ref[...] + c_ref[...]


def _final_add(self_out, p0, p1, n, d):
    grid = n // BM
    return pl.pallas_call(
        _add_body,
        grid=(grid,),
        in_specs=[
            pl.BlockSpec((BM, d), lambda i: (i, 0)),
            pl.BlockSpec((BM, d), lambda i: (i, 0)),
            pl.BlockSpec((BM, d), lambda i: (i, 0)),
        ],
        out_specs=pl.BlockSpec((BM, d), lambda i: (i, 0)),
        out_shape=jax.ShapeDtypeStruct((n, d), jnp.float32),
    )(self_out, p0, p1)


def kernel(x, edge_index, edge_types, rel_W, rel_b, self_W, self_b,
           att_W, att_b):
    n, d = x.shape
    r = rel_W.shape[0]
    e = edge_index.shape[1]

    ws = self_W.T
    bs = self_b.reshape(1, d)
    wr = jnp.transpose(rel_W, (2, 0, 1)).reshape(d, r * d)
    br = rel_b.reshape(1, r * d)
    wa = jnp.zeros((d, 8), jnp.float32)
    wa = wa.at[:, 0].set(att_W[0, :d]).at[:, 1].set(att_W[0, d:])
    ba = jnp.zeros((1, 8), jnp.float32).at[0, 0].set(att_b[0])

    self_out, yc, av = _dense_stage(x, ws, bs, wr, br, wa, ba)
    y = yc.reshape(n * r, d)
    a_src = av[:, 0]
    a_tgt = av[:, 1]

    ring = NSLOT * CHUNK
    per_worker = -(-e // (NW * ring)) * ring
    ep = per_worker * NW
    n_acc = -(-(n + LANES) // 128) * 128
    pad = ep - e
    src = edge_index[0].astype(jnp.int32)
    tgt = edge_index[1].astype(jnp.int32)
    typ = edge_types.astype(jnp.int32)
    if pad:
        src = jnp.concatenate([src, jnp.zeros((pad,), jnp.int32)])
        tgt = jnp.concatenate([tgt, jnp.full((pad,), n, jnp.int32)])
        typ = jnp.concatenate([typ, jnp.zeros((pad,), jnp.int32)])
    epk = jnp.concatenate(
        [src.reshape(-1, 1, CHUNK), tgt.reshape(-1, 1, CHUNK),
         typ.reshape(-1, 1, CHUNK)], axis=1).reshape(-1)
    zeros = jnp.zeros((n_acc, d), jnp.float32)

    partial = _sparse_stage(y, a_src, a_tgt, epk, zeros,
                            n, n_acc, r, per_worker)
    p0 = lax.slice(partial, (0, 0), (n, d))
    p1 = lax.slice(partial, (n_acc, 0), (n_acc + n, d))
    return _final_add(self_out, p0, p1, n, d)


# speedup vs baseline: 1.3687x; 1.2540x over previous
import functools
import math

import jax
import jax.numpy as jnp
from jax import lax
from jax.experimental import pallas as pl
from jax.experimental.pallas import tpu as pltpu
from jax.experimental.pallas import tpu_sc as plsc

LANES = 16
CHUNK = 48
NSLOT = 3
NW = 32
BM = 400


def _dense_body(x_ref, ws_ref, bs_ref, wr_ref, br_ref, wa_ref, ba_ref,
                self_ref, yc_ref, av_ref):
    xb = x_ref[...]
    self_ref[...] = (
        jnp.dot(xb, ws_ref[...], preferred_element_type=jnp.float32)
        + bs_ref[...])
    yc_ref[...] = (
        jnp.dot(xb, wr_ref[...], preferred_element_type=jnp.float32)
        + br_ref[...])
    av_ref[...] = (
        jnp.dot(xb, wa_ref[...], preferred_element_type=jnp.float32)
        + ba_ref[...])


def _dense_stage(x, ws, bs, wr, br, wa, ba):
    n, d = x.shape
    rd = wr.shape[1]
    grid = n // BM
    return pl.pallas_call(
        _dense_body,
        grid=(grid,),
        in_specs=[
            pl.BlockSpec((BM, d), lambda i: (i, 0)),
            pl.BlockSpec((d, d), lambda i: (0, 0)),
            pl.BlockSpec((1, d), lambda i: (0, 0)),
            pl.BlockSpec((d, rd), lambda i: (0, 0)),
            pl.BlockSpec((1, rd), lambda i: (0, 0)),
            pl.BlockSpec((d, 8), lambda i: (0, 0)),
            pl.BlockSpec((1, 8), lambda i: (0, 0)),
        ],
        out_specs=[
            pl.BlockSpec((BM, d), lambda i: (i, 0)),
            pl.BlockSpec((BM, rd), lambda i: (i, 0)),
            pl.BlockSpec((BM, 8), lambda i: (i, 0)),
        ],
        out_shape=[
            jax.ShapeDtypeStruct((n, d), jnp.float32),
            jax.ShapeDtypeStruct((n, rd), jnp.float32),
            jax.ShapeDtypeStruct((n, 8), jnp.float32),
        ],
    )(x, ws, bs, wr, br, wa, ba)


def _sc_body(n_nodes, n_acc, r_rel, per_worker,
             y_hbm, asrc_hbm, atgt_hbm, src_hbm, tgt_hbm, typ_hbm, zeros_hbm,
             partial_hbm,
             acc, asrc_v, atgt_v, src_v, tgt_v, typ_v,
             g_v, tc_v, att_v, rows_v,
             esem0, esem1, esem2, gsem0, gsem1, gsem2, ssem0, ssem1, ssem2):
    esem = (esem0, esem1, esem2)
    gsem = (gsem0, gsem1, gsem2)
    ssem = (ssem0, ssem1, ssem2)
    cid = lax.axis_index("c")
    sid = lax.axis_index("s")
    wid = cid * 16 + sid
    stripe = n_acc // 16
    n_chunks = per_worker // CHUNK
    n_groups = n_chunks // NSLOT
    base_w = wid * per_worker

    pltpu.sync_copy(zeros_hbm.at[pl.ds(sid * stripe, stripe)],
                    acc.at[pl.ds(sid * stripe, stripe)])

    pltpu.sync_copy(asrc_hbm, asrc_v)
    pltpu.sync_copy(atgt_hbm, atgt_v)
    plsc.subcore_barrier()

    def fire_e(c, s):
        base = base_w + c * CHUNK
        pltpu.async_copy(src_hbm.at[pl.ds(base, CHUNK)], src_v.at[s], esem[s])
        pltpu.async_copy(tgt_hbm.at[pl.ds(base, CHUNK)], tgt_v.at[s], esem[s])
        pltpu.async_copy(typ_hbm.at[pl.ds(base, CHUNK)], typ_v.at[s], esem[s])

    def wait_e(c, s):
        base = base_w + c * CHUNK
        pltpu.make_async_copy(src_hbm.at[pl.ds(base, CHUNK)], src_v.at[s],
                              esem[s]).wait()
        pltpu.make_async_copy(tgt_hbm.at[pl.ds(base, CHUNK)], tgt_v.at[s],
                              esem[s]).wait()
        pltpu.make_async_copy(typ_hbm.at[pl.ds(base, CHUNK)], typ_v.at[s],
                              esem[s]).wait()

    def prep(s):
        for k in range(CHUNK // LANES):
            sl = pl.ds(k * LANES, LANES)
            s16 = src_v[s, sl]
            t16 = tgt_v[s, sl]
            g_v[s, sl] = s16 * r_rel + typ_v[s, sl]
            tc_v[s, sl] = t16
            za = plsc.load_gather(asrc_v, [s16])
            zb = plsc.load_gather(atgt_v, [t16])
            att_v[pl.ds(s * CHUNK + k * LANES, LANES)] = (
                1.0 / (1.0 + jnp.exp(-(za + zb))))

    def fire_g(s):
        pltpu.async_copy(y_hbm.at[g_v.at[s]], rows_v.at[s], gsem[s])

    def wait_g(s):
        pltpu.make_async_copy(y_hbm.at[g_v.at[s]], rows_v.at[s],
                              gsem[s]).wait()

    def scale(s):
        def scale_body(e, carry):
            for u in range(2):
                eu = e * 2 + u
                a = plsc.load_gather(
                    att_v, [jnp.full((LANES,), s * CHUNK + eu, jnp.int32)])
                for dd in range(128 // LANES):
                    sl = pl.ds(dd * LANES, LANES)
                    rows_v[s, eu, sl] = rows_v[s, eu, sl] * a
            return carry

        lax.fori_loop(0, CHUNK // 2, scale_body, 0)

    def fire_w(s):
        pltpu.async_copy(rows_v.at[s], acc.at[tc_v.at[s]], ssem[s], add=True)

    def wait_w(s):
        pltpu.make_async_copy(rows_v.at[s], acc.at[tc_v.at[s]],
                              ssem[s]).wait()

    for s in range(NSLOT):
        fire_e(s, s)
    wait_e(0, 0)
    prep(0)
    fire_g(0)

    def group_body(j, carry):
        for s in range(NSLOT):
            c = j * NSLOT + s
            ns = (s + 1) % NSLOT
            if s == 2:
                wait_w(ns)
            else:
                @pl.when(j >= 1)
                def _retire(ns=ns):
                    wait_w(ns)
            if s == 2:
                @pl.when(j < n_groups - 1)
                def _stage(c=c, ns=ns):
                    wait_e(c + 1, ns)
                    prep(ns)
                    fire_g(ns)
            else:
                wait_e(c + 1, ns)
                prep(ns)
                fire_g(ns)
            @pl.when(j < n_groups - 1)
            def _prefetch(c=c, s=s):
                fire_e(c + NSLOT, s)
            wait_g(s)
            scale(s)
            fire_w(s)
        return carry

    lax.fori_loop(0, n_groups, group_body, 0)
    wait_w((n_chunks - 2) % NSLOT)
    wait_w((n_chunks - 1) % NSLOT)
    plsc.subcore_barrier()

    dst0 = cid * n_acc + sid * stripe
    pltpu.sync_copy(acc.at[pl.ds(sid * stripe, stripe)],
                    partial_hbm.at[pl.ds(dst0, stripe)])


def _sparse_stage(y, a_src, a_tgt, src, tgt, typ, zeros, n_nodes, n_acc,
                  r_rel, per_worker):
    d = y.shape[1]
    mesh = plsc.VectorSubcoreMesh(core_axis_name="c", subcore_axis_name="s",
                                  num_cores=2, num_subcores=16)
    body = functools.partial(_sc_body, n_nodes, n_acc, r_rel, per_worker)
    return pl.kernel(
        body,
        out_type=jax.ShapeDtypeStruct((2 * n_acc, d), jnp.float32),
        mesh=mesh,
        scratch_types=[
            pltpu.VMEM_SHARED((n_acc, d), jnp.float32),
            pltpu.VMEM((n_nodes,), jnp.float32),
            pltpu.VMEM((n_nodes,), jnp.float32),
            pltpu.VMEM((NSLOT, CHUNK), jnp.int32),
            pltpu.VMEM((NSLOT, CHUNK), jnp.int32),
            pltpu.VMEM((NSLOT, CHUNK), jnp.int32),
            pltpu.VMEM((NSLOT, CHUNK), jnp.int32),
            pltpu.VMEM((NSLOT, CHUNK), jnp.int32),
            pltpu.VMEM((NSLOT * CHUNK,), jnp.float32),
            pltpu.VMEM((NSLOT, CHUNK, d), jnp.float32),
            pltpu.SemaphoreType.DMA,
            pltpu.SemaphoreType.DMA,
            pltpu.SemaphoreType.DMA,
            pltpu.SemaphoreType.DMA,
            pltpu.SemaphoreType.DMA,
            pltpu.SemaphoreType.DMA,
            pltpu.SemaphoreType.DMA,
            pltpu.SemaphoreType.DMA,
            pltpu.SemaphoreType.DMA,
        ],
        compiler_params=pltpu.CompilerParams(needs_layout_passes=False),
    )(y, a_src, a_tgt, src, tgt, typ, zeros)


def _add_body(a_ref, b_ref, c_ref, o_ref):
    o_ref[...] = a_ref[...] + b_ref[...] + c_ref[...]


def _final_add(self_out, p0, p1, n, d):
    grid = n // BM
    return pl.pallas_call(
        _add_body,
        grid=(grid,),
        in_specs=[
            pl.BlockSpec((BM, d), lambda i: (i, 0)),
            pl.BlockSpec((BM, d), lambda i: (i, 0)),
            pl.BlockSpec((BM, d), lambda i: (i, 0)),
        ],
        out_specs=pl.BlockSpec((BM, d), lambda i: (i, 0)),
        out_shape=jax.ShapeDtypeStruct((n, d), jnp.float32),
    )(self_out, p0, p1)


def kernel(x, edge_index, edge_types, rel_W, rel_b, self_W, self_b,
           att_W, att_b):
    n, d = x.shape
    r = rel_W.shape[0]
    e = edge_index.shape[1]

    ws = self_W.T
    bs = self_b.reshape(1, d)
    wr = jnp.transpose(rel_W, (2, 0, 1)).reshape(d, r * d)
    br = rel_b.reshape(1, r * d)
    wa = jnp.zeros((d, 8), jnp.float32)
    wa = wa.at[:, 0].set(att_W[0, :d]).at[:, 1].set(att_W[0, d:])
    ba = jnp.zeros((1, 8), jnp.float32).at[0, 0].set(att_b[0])

    self_out, yc, av = _dense_stage(x, ws, bs, wr, br, wa, ba)
    y = yc.reshape(n * r, d)
    a_src = av[:, 0]
    a_tgt = av[:, 1]

    ring = NSLOT * CHUNK
    per_worker = -(-e // (NW * ring)) * ring
    ep = per_worker * NW
    n_acc = -(-(n + LANES) // 128) * 128
    pad = ep - e
    src = edge_index[0].astype(jnp.int32)
    tgt = edge_index[1].astype(jnp.int32)
    typ = edge_types.astype(jnp.int32)
    if pad:
        src = jnp.concatenate([src, jnp.zeros((pad,), jnp.int32)])
        tgt = jnp.concatenate([tgt, jnp.full((pad,), n, jnp.int32)])
        typ = jnp.concatenate([typ, jnp.zeros((pad,), jnp.int32)])
    zeros = jnp.zeros((n_acc, d), jnp.float32)

    partial = _sparse_stage(y, a_src, a_tgt, src, tgt, typ, zeros,
                            n, n_acc, r, per_worker)
    p0 = lax.slice(partial, (0, 0), (n, d))
    p1 = lax.slice(partial, (n_acc, 0), (n_acc + n, d))
    return _final_add(self_out, p0, p1, n, d)
